# baseline (speedup 1.0000x reference)
import jax
import jax.numpy as jnp
from jax.experimental import pallas as pl

_NUM_SEGMENTS = 10000


def _body(d_ref, m_ref, s_ref, o_ref):
    o_ref[...] = jnp.exp(d_ref[...] - m_ref[...]) / s_ref[...]


def kernel(data, segment_ids):
    seg_max = jax.ops.segment_max(data, segment_ids, num_segments=_NUM_SEGMENTS)
    seg_max = jnp.where(jnp.isfinite(seg_max), seg_max, 0.0)
    shifted_max = seg_max[segment_ids]
    e = jnp.exp(data - shifted_max)
    denom = jax.ops.segment_sum(e, segment_ids, num_segments=_NUM_SEGMENTS)
    denom_g = denom[segment_ids]

    n, f = data.shape
    blk = 2000
    out = pl.pallas_call(
        _body,
        grid=(n // blk,),
        in_specs=[
            pl.BlockSpec((blk, f), lambda i: (i, 0)),
            pl.BlockSpec((blk, f), lambda i: (i, 0)),
            pl.BlockSpec((blk, f), lambda i: (i, 0)),
        ],
        out_specs=pl.BlockSpec((blk, f), lambda i: (i, 0)),
        out_shape=jax.ShapeDtypeStruct((n, f), data.dtype),
    )(data, shifted_max, denom_g)
    return out


# speedup vs baseline: 3.4302x; 3.4302x over previous
import dataclasses
import functools

import jax
import jax.numpy as jnp
from jax import lax
from jax.experimental import pallas as pl
from jax.experimental.pallas import tpu as pltpu
from jax.experimental.pallas import tpu_sc as plsc

N = 320000
F = 128
NSEG = 10000
NV = F // 16
NW = 32
SPW = 313
G = 16
NGROUPS = (SPW + G - 1) // G
C = 768
ST_LEN = 328
SPAD = 10248


def _copy_rows(src, dst, src_row, dst_row, n, max_log2):
    for k in range(max_log2, -1, -1):
        sz = 1 << k
        off = lax.shift_left(lax.shift_right_logical(n, k + 1), k + 1)

        @pl.when((n & sz) != 0)
        def _():
            pltpu.sync_copy(
                src.at[pl.ds((src_row + off) * F, sz * F)],
                dst.at[pl.ds((dst_row + off) * F, sz * F)],
            )


def _sc_softmax(data_flat, starts):
    mesh = plsc.VectorSubcoreMesh(core_axis_name="c", subcore_axis_name="s")
    cp = pltpu.CompilerParams()
    if "needs_layout_passes" in pltpu.CompilerParams.__dataclass_fields__:
        cp = dataclasses.replace(cp, needs_layout_passes=False)

    @functools.partial(
        pl.kernel,
        out_type=jax.ShapeDtypeStruct((N * F,), jnp.float32),
        mesh=mesh,
        compiler_params=cp,
        scratch_types=[
            pltpu.VMEM((C * F,), jnp.float32),
            pltpu.VMEM((ST_LEN,), jnp.int32),
        ],
    )
    def body(data_hbm, starts_hbm, out_hbm, buf, st_v):
        w = lax.axis_index("s") * 2 + lax.axis_index("c")
        sbase = w * SPW
        abase = (sbase // 8) * 8
        slo = sbase - abase
        pltpu.sync_copy(starts_hbm.at[pl.ds(abase, ST_LEN)], st_v)

        def st_read(idx):
            vec = plsc.load_gather(st_v, [jnp.full((16,), idx, jnp.int32)])
            return jnp.max(vec)

        zero8 = (jnp.zeros((16,), jnp.float32),) * NV
        minf8 = (jnp.full((16,), -jnp.inf, jnp.float32),) * NV

        def seg_compute(lo, n):
            def max_body(r, m):
                base = (lo + r) * F
                return tuple(
                    jnp.maximum(m[v], buf[pl.ds(base + v * 16, 16)])
                    for v in range(NV)
                )

            m = lax.fori_loop(0, n, max_body, minf8)

            def exp_body(r, s):
                base = (lo + r) * F
                out = []
                for v in range(NV):
                    e = jnp.exp(buf[pl.ds(base + v * 16, 16)] - m[v])
                    buf[pl.ds(base + v * 16, 16)] = e
                    out.append(s[v] + e)
                return tuple(out)

            ssum = lax.fori_loop(0, n, exp_body, zero8)
            rcp = tuple(1.0 / ssum[v] for v in range(NV))

            def scale_body(r, carry):
                base = (lo + r) * F
                for v in range(NV):
                    buf[pl.ds(base + v * 16, 16)] = (
                        buf[pl.ds(base + v * 16, 16)] * rcp[v]
                    )
                return carry

            lax.fori_loop(0, n, scale_body, 0)

        def seg_stream(st, en):
            n = en - st
            nch = (n + C - 1) // C

            def chunk_a(i, m):
                base = st + i * C
                csz = jnp.minimum(C, en - base)
                _copy_rows(data_hbm, buf, base, 0, csz, 9)

                def mb(r, mm):
                    rb = r * F
                    return tuple(
                        jnp.maximum(mm[v], buf[pl.ds(rb + v * 16, 16)])
                        for v in range(NV)
                    )

                return lax.fori_loop(0, csz, mb, m)

            m = lax.fori_loop(0, nch, chunk_a, minf8)

            def chunk_b(i, s):
                base = st + i * C
                csz = jnp.minimum(C, en - base)
                _copy_rows(data_hbm, buf, base, 0, csz, 9)

                def eb(r, ss):
                    rb = r * F
                    out = []
                    for v in range(NV):
                        e = jnp.exp(buf[pl.ds(rb + v * 16, 16)] - m[v])
                        buf[pl.ds(rb + v * 16, 16)] = e
                        out.append(ss[v] + e)
                    return tuple(out)

                s2 = lax.fori_loop(0, csz, eb, s)
                _copy_rows(buf, out_hbm, 0, base, csz, 9)
                return s2

            ssum = lax.fori_loop(0, nch, chunk_b, zero8)
            rcp = tuple(1.0 / ssum[v] for v in range(NV))

            def chunk_c(i, carry):
                base = st + i * C
                csz = jnp.minimum(C, en - base)
                _copy_rows(out_hbm, buf, base, 0, csz, 9)

                def sb(r, cc):
                    rb = r * F
                    for v in range(NV):
                        buf[pl.ds(rb + v * 16, 16)] = (
                            buf[pl.ds(rb + v * 16, 16)] * rcp[v]
                        )
                    return cc

                lax.fori_loop(0, csz, sb, 0)
                _copy_rows(buf, out_hbm, 0, base, csz, 9)
                return carry

            lax.fori_loop(0, nch, chunk_c, 0)

        @pl.loop(0, NGROUPS)
        def _(g):
            sidx = slo + g * G
            eidx = jnp.minimum(sidx + G, slo + SPW)
            row0 = st_read(sidx)
            row1 = st_read(eidx)
            ng = row1 - row0

            @pl.when((ng > 0) & (ng <= C))
            def _():
                _copy_rows(data_hbm, buf, row0, 0, ng, 9)

                @pl.loop(0, G)
                def _(j):
                    st = st_read(sidx + j)
                    en = st_read(sidx + j + 1)

                    @pl.when((sidx + j < eidx) & (en > st))
                    def _():
                        seg_compute(st - row0, en - st)

                _copy_rows(buf, out_hbm, 0, row0, ng, 9)

            @pl.when(ng > C)
            def _():
                @pl.loop(0, G)
                def _(j):
                    st = st_read(sidx + j)
                    en = st_read(sidx + j + 1)

                    @pl.when((sidx + j < eidx) & (en > st))
                    def _():
                        seg_stream(st, en)

    return body(data_flat, starts)


@jax.jit
def kernel(data, segment_ids):
    ids = segment_ids.astype(jnp.int32)
    queries = jnp.arange(SPAD, dtype=jnp.int32)
    starts = jnp.searchsorted(ids, queries, side="left").astype(jnp.int32)
    out_flat = _sc_softmax(data.reshape(-1), starts)
    return out_flat.reshape(N, F)


# speedup vs baseline: 9.7079x; 2.8302x over previous
import dataclasses
import functools

import jax
import jax.numpy as jnp
from jax import lax
from jax.experimental import pallas as pl
from jax.experimental.pallas import tpu as pltpu
from jax.experimental.pallas import tpu_sc as plsc

N = 320000
F = 128
NV = F // 16
NW = 32
R = N // NW
G = 8
C = 384
CH = 256
EXT = 2048
SLAB = R + EXT + 16
IDS_PAD = 8 + N + 2112
BIG = 1 << 30


def _pieces(n, max_log2):
    for k in range(max_log2, -1, -1):
        sz = 1 << k
        off = lax.shift_left(lax.shift_right_logical(n, k + 1), k + 1)
        yield sz, off


def _issue_rows(src, dst, src_row, dst_row, n, max_log2, sem):
    for sz, off in _pieces(n, max_log2):
        @pl.when((n & sz) != 0)
        def _():
            pltpu.async_copy(
                src.at[pl.ds((src_row + off) * F, sz * F)],
                dst.at[pl.ds((dst_row + off) * F, sz * F)],
                sem,
            )


def _wait_rows(src, dst, src_row, dst_row, n, max_log2, sem):
    for sz, off in _pieces(n, max_log2):
        @pl.when((n & sz) != 0)
        def _():
            pltpu.make_async_copy(
                src.at[pl.ds((src_row + off) * F, sz * F)],
                dst.at[pl.ds((dst_row + off) * F, sz * F)],
                sem,
            ).wait()


def _copy_rows(src, dst, src_row, dst_row, n, max_log2, sem):
    _issue_rows(src, dst, src_row, dst_row, n, max_log2, sem)
    _wait_rows(src, dst, src_row, dst_row, n, max_log2, sem)


def _sc_softmax(data_flat, ids_pad):
    mesh = plsc.VectorSubcoreMesh(core_axis_name="c", subcore_axis_name="s")
    cp = pltpu.CompilerParams()
    if "needs_layout_passes" in pltpu.CompilerParams.__dataclass_fields__:
        cp = dataclasses.replace(cp, needs_layout_passes=False)

    @functools.partial(
        pl.kernel,
        out_type=jax.ShapeDtypeStruct((N * F,), jnp.float32),
        mesh=mesh,
        compiler_params=cp,
        scratch_types=[
            pltpu.VMEM((C * F,), jnp.float32),
            pltpu.VMEM((C * F,), jnp.float32),
            pltpu.VMEM((SLAB,), jnp.int32),
            pltpu.VMEM((R + 16,), jnp.int32),
            pltpu.SemaphoreType.DMA,
            pltpu.SemaphoreType.DMA,
            pltpu.SemaphoreType.DMA,
            pltpu.SemaphoreType.DMA,
        ],
    )
    def body(data_hbm, ids_hbm, out_hbm, bufa, bufb, slab, bd_v,
             lsem0, lsem1, ssem, dsem):
        w = lax.axis_index("s") * 2 + lax.axis_index("c")
        base = pl.multiple_of(w * R, 8)
        lane = jax.lax.broadcasted_iota(jnp.int32, (16,), 0)

        def scalar_of(vec):
            return jnp.max(vec)

        def bd_read(idx):
            vec = plsc.load_gather(bd_v, [jnp.full((16,), idx, jnp.int32)])
            return jnp.max(vec)

        pltpu.sync_copy(ids_hbm.at[pl.ds(base, SLAB)], slab)

        def scan_body(i, cnt):
            vec = slab[pl.ds(8 + 16 * i, 16)]
            prev = slab[pl.ds(7 + 16 * i, 16)]
            flags = vec != prev
            rows = (base + 16 * i) + lane
            plsc.store_compressed(bd_v.at[pl.ds(cnt, 16)], rows, mask=flags)
            return cnt + scalar_of(plsc.all_reduce_population_count(flags))

        ns = lax.fori_loop(0, R // 16, scan_body, jnp.int32(0))

        def ext_body(i, end):
            vec = slab[pl.ds(8 + R + 16 * i, 16)]
            prev = slab[pl.ds(7 + R + 16 * i, 16)]
            flags = vec != prev
            first = scalar_of(plsc.all_reduce_ffs(flags))
            cand = base + R + 16 * i + first
            return jnp.minimum(end, jnp.where(first < 16, cand, jnp.int32(BIG)))

        end0 = lax.fori_loop(0, EXT // 16, ext_body, jnp.int32(BIG))

        def while_cond(carry):
            pos, end = carry
            return end >= BIG

        def while_body(carry):
            pos, end = carry
            posa = pl.multiple_of(pos, 8)
            pltpu.sync_copy(ids_hbm.at[pl.ds(posa, 520)], slab.at[pl.ds(0, 520)])

            def wscan(i, e):
                vec = slab[pl.ds(8 + 16 * i, 16)]
                prev = slab[pl.ds(7 + 16 * i, 16)]
                flags = vec != prev
                first = scalar_of(plsc.all_reduce_ffs(flags))
                cand = pos + 16 * i + first
                return jnp.minimum(e, jnp.where(first < 16, cand, jnp.int32(BIG)))

            return pos + 512, lax.fori_loop(0, 32, wscan, end)

        _, end_w = lax.while_loop(
            while_cond, while_body, (jnp.int32(base + R + EXT), end0)
        )

        plsc.store_scatter(
            bd_v, [jnp.full((16,), ns, jnp.int32)],
            jnp.full((16,), end_w, jnp.int32), mask=lane == 0,
        )

        zero8 = (jnp.zeros((16,), jnp.float32),) * NV
        minf8 = (jnp.full((16,), -jnp.inf, jnp.float32),) * NV

        def seg_compute(bufx, lo, n):
            def max_body(r, m):
                rb = (lo + r) * F
                return tuple(
                    jnp.maximum(m[v], bufx[pl.ds(rb + v * 16, 16)])
                    for v in range(NV)
                )

            m = lax.fori_loop(0, n, max_body, minf8)

            def exp_body(r, s):
                rb = (lo + r) * F
                out = []
                for v in range(NV):
                    e = jnp.exp(bufx[pl.ds(rb + v * 16, 16)] - m[v])
                    bufx[pl.ds(rb + v * 16, 16)] = e
                    out.append(s[v] + e)
                return tuple(out)

            ssum = lax.fori_loop(0, n, exp_body, zero8)
            rcp = tuple(1.0 / ssum[v] for v in range(NV))

            def scale_body(r, carry):
                rb = (lo + r) * F
                for v in range(NV):
                    bufx[pl.ds(rb + v * 16, 16)] = (
                        bufx[pl.ds(rb + v * 16, 16)] * rcp[v]
                    )
                return carry

            lax.fori_loop(0, n, scale_body, 0)

        def seg_stream(bufx, st, en):
            n = en - st
            nch = (n + CH - 1) // CH

            def chunk_a(i, m):
                cb = st + i * CH
                csz = jnp.minimum(CH, en - cb)
                _copy_rows(data_hbm, bufx, cb, 0, csz, 9, dsem)

                def mb(r, mm):
                    rb = r * F
                    return tuple(
                        jnp.maximum(mm[v], bufx[pl.ds(rb + v * 16, 16)])
                        for v in range(NV)
                    )

                return lax.fori_loop(0, csz, mb, m)

            m = lax.fori_loop(0, nch, chunk_a, minf8)

            def chunk_b(i, s):
                cb = st + i * CH
                csz = jnp.minimum(CH, en - cb)
                _copy_rows(data_hbm, bufx, cb, 0, csz, 9, dsem)

                def eb(r, ss):
                    rb = r * F
                    out = []
                    for v in range(NV):
                        e = jnp.exp(bufx[pl.ds(rb + v * 16, 16)] - m[v])
                        bufx[pl.ds(rb + v * 16, 16)] = e
                        out.append(ss[v] + e)
                    return tuple(out)

                s2 = lax.fori_loop(0, csz, eb, s)
                _copy_rows(bufx, out_hbm, 0, cb, csz, 9, dsem)
                return s2

            ssum = lax.fori_loop(0, nch, chunk_b, zero8)
            rcp = tuple(1.0 / ssum[v] for v in range(NV))

            def chunk_c(i, carry):
                cb = st + i * CH
                csz = jnp.minimum(CH, en - cb)
                _copy_rows(out_hbm, bufx, cb, 0, csz, 9, dsem)

                def sb(r, cc):
                    rb = r * F
                    for v in range(NV):
                        bufx[pl.ds(rb + v * 16, 16)] = (
                            bufx[pl.ds(rb + v * 16, 16)] * rcp[v]
                        )
                    return cc

                lax.fori_loop(0, csz, sb, 0)
                _copy_rows(bufx, out_hbm, 0, cb, csz, 9, dsem)
                return carry

            lax.fori_loop(0, nch, chunk_c, 0)

        ngroups = (ns + G - 1) // G

        def g_info(g):
            kidx = g * G
            kend = jnp.minimum(kidx + G, ns)
            return kidx, kend, bd_read(kidx), bd_read(kend)

        def compute_fast(g, bufx, row0):
            kidx, kend, _, _ = g_info(g)

            @pl.loop(0, G)
            def _(j):
                @pl.when(kidx + j < kend)
                def _():
                    st = bd_read(kidx + j)
                    en = bd_read(kidx + j + 1)
                    seg_compute(bufx, st - row0, en - st)

        def fallback(g, bufx):
            kidx, kend, _, _ = g_info(g)

            @pl.loop(0, G)
            def _(j):
                @pl.when(kidx + j < kend)
                def _():
                    st = bd_read(kidx + j)
                    en = bd_read(kidx + j + 1)
                    seg_stream(bufx, st, en)

        npairs = (ngroups + 1) // 2

        @pl.loop(0, npairs)
        def _(p):
            g0 = p * 2
            g1 = g0 + 1
            _, _, r00, r01 = g_info(g0)
            ng0 = r01 - r00
            in1 = g1 < ngroups
            _, _, r10, r11 = g_info(g1)
            ng1 = r11 - r10
            ld1 = in1 & (ng1 <= C)

            @pl.when(ng0 <= C)
            def _():
                _issue_rows(data_hbm, bufa, r00, 0, ng0, 9, lsem0)

            @pl.when(ld1)
            def _():
                _issue_rows(data_hbm, bufb, r10, 0, ng1, 9, lsem1)

            @pl.when(ng0 <= C)
            def _():
                _wait_rows(data_hbm, bufa, r00, 0, ng0, 9, lsem0)
                compute_fast(g0, bufa, r00)
                _issue_rows(bufa, out_hbm, 0, r00, ng0, 9, ssem)

            @pl.when(ng0 > C)
            def _():
                fallback(g0, bufa)

            @pl.when(ld1)
            def _():
                _wait_rows(data_hbm, bufb, r10, 0, ng1, 9, lsem1)
                compute_fast(g1, bufb, r10)

            @pl.when(ng0 <= C)
            def _():
                _wait_rows(bufa, out_hbm, 0, r00, ng0, 9, ssem)

            @pl.when(ld1)
            def _():
                _copy_rows(bufb, out_hbm, 0, r10, ng1, 9, dsem)

            @pl.when(in1 & (ng1 > C))
            def _():
                fallback(g1, bufb)

    return body(data_flat, ids_pad)


@jax.jit
def kernel(data, segment_ids):
    ids = segment_ids.astype(jnp.int32)
    head = jnp.full((8,), -1, jnp.int32)
    tail = jnp.full((IDS_PAD - 8 - N,), BIG, jnp.int32)
    ids_pad = jnp.concatenate([head, ids, tail])
    out_flat = _sc_softmax(data.reshape(-1), ids_pad)
    return out_flat.reshape(N, F)


# speedup vs baseline: 10.5772x; 1.0895x over previous
import dataclasses
import functools

import jax
import jax.numpy as jnp
from jax import lax
from jax.experimental import pallas as pl
from jax.experimental.pallas import tpu as pltpu
from jax.experimental.pallas import tpu_sc as plsc

N = 320000
F = 128
NV = F // 16
NW = 32
R = N // NW
G = 8
C = 384
CH = 256
EXT = 2048
SLAB = R + EXT + 16
IDS_PAD = 8 + N + 2112
BIG = 1 << 30


def _pieces(n, max_log2):
    for k in range(max_log2, -1, -1):
        sz = 1 << k
        off = lax.shift_left(lax.shift_right_logical(n, k + 1), k + 1)
        yield sz, off


def _issue_rows(src, dst, src_row, dst_row, n, max_log2, sem):
    for sz, off in _pieces(n, max_log2):
        @pl.when((n & sz) != 0)
        def _():
            pltpu.async_copy(
                src.at[pl.ds((src_row + off) * F, sz * F)],
                dst.at[pl.ds((dst_row + off) * F, sz * F)],
                sem,
            )


def _wait_rows(src, dst, src_row, dst_row, n, max_log2, sem):
    for sz, off in _pieces(n, max_log2):
        @pl.when((n & sz) != 0)
        def _():
            pltpu.make_async_copy(
                src.at[pl.ds((src_row + off) * F, sz * F)],
                dst.at[pl.ds((dst_row + off) * F, sz * F)],
                sem,
            ).wait()


def _copy_rows(src, dst, src_row, dst_row, n, max_log2, sem):
    _issue_rows(src, dst, src_row, dst_row, n, max_log2, sem)
    _wait_rows(src, dst, src_row, dst_row, n, max_log2, sem)


def _sc_softmax(data_flat, ids_pad):
    mesh = plsc.VectorSubcoreMesh(core_axis_name="c", subcore_axis_name="s")
    cp = pltpu.CompilerParams()
    if "needs_layout_passes" in pltpu.CompilerParams.__dataclass_fields__:
        cp = dataclasses.replace(cp, needs_layout_passes=False)

    @functools.partial(
        pl.kernel,
        out_type=jax.ShapeDtypeStruct((N * F,), jnp.float32),
        mesh=mesh,
        compiler_params=cp,
        scratch_types=[
            pltpu.VMEM((C * F,), jnp.float32),
            pltpu.VMEM((C * F,), jnp.float32),
            pltpu.VMEM((SLAB,), jnp.int32),
            pltpu.VMEM((R + 16,), jnp.int32),
            pltpu.SemaphoreType.DMA,
            pltpu.SemaphoreType.DMA,
            pltpu.SemaphoreType.DMA,
            pltpu.SemaphoreType.DMA,
        ],
    )
    def body(data_hbm, ids_hbm, out_hbm, bufa, bufb, slab, bd_v,
             lsem0, lsem1, ssem, dsem):
        w = lax.axis_index("s") * 2 + lax.axis_index("c")
        base = pl.multiple_of(w * R, 8)
        lane = jax.lax.broadcasted_iota(jnp.int32, (16,), 0)

        def scalar_of(vec):
            return jnp.max(vec)

        def bd_read(idx):
            vec = plsc.load_gather(bd_v, [jnp.full((16,), idx, jnp.int32)])
            return jnp.max(vec)

        pltpu.sync_copy(ids_hbm.at[pl.ds(base, SLAB)], slab)

        def scan_body(i, cnt):
            vec = slab[pl.ds(8 + 16 * i, 16)]
            prev = slab[pl.ds(7 + 16 * i, 16)]
            flags = vec != prev
            rows = (base + 16 * i) + lane
            plsc.store_compressed(bd_v.at[pl.ds(cnt, 16)], rows, mask=flags)
            return cnt + scalar_of(plsc.all_reduce_population_count(flags))

        ns = lax.fori_loop(0, R // 16, scan_body, jnp.int32(0))

        def ext_body(i, end):
            vec = slab[pl.ds(8 + R + 16 * i, 16)]
            prev = slab[pl.ds(7 + R + 16 * i, 16)]
            flags = vec != prev
            first = scalar_of(plsc.all_reduce_ffs(flags))
            cand = base + R + 16 * i + first
            return jnp.minimum(end, jnp.where(first < 16, cand, jnp.int32(BIG)))

        end0 = lax.fori_loop(0, EXT // 16, ext_body, jnp.int32(BIG))

        def while_cond(carry):
            pos, end = carry
            return end >= BIG

        def while_body(carry):
            pos, end = carry
            posa = pl.multiple_of(pos, 8)
            pltpu.sync_copy(ids_hbm.at[pl.ds(posa, 520)], slab.at[pl.ds(0, 520)])

            def wscan(i, e):
                vec = slab[pl.ds(8 + 16 * i, 16)]
                prev = slab[pl.ds(7 + 16 * i, 16)]
                flags = vec != prev
                first = scalar_of(plsc.all_reduce_ffs(flags))
                cand = pos + 16 * i + first
                return jnp.minimum(e, jnp.where(first < 16, cand, jnp.int32(BIG)))

            return pos + 512, lax.fori_loop(0, 32, wscan, end)

        _, end_w = lax.while_loop(
            while_cond, while_body, (jnp.int32(base + R + EXT), end0)
        )

        plsc.store_scatter(
            bd_v, [jnp.full((16,), ns, jnp.int32)],
            jnp.full((16,), end_w, jnp.int32), mask=lane == 0,
        )

        zero8 = (jnp.zeros((16,), jnp.float32),) * NV
        minf8 = (jnp.full((16,), -jnp.inf, jnp.float32),) * NV

        def seg_compute(bufx, lo, n):
            nh = n // 2
            odd = (n & 1) == 1
            rbt = (lo + n - 1) * F

            def max_body(r, m):
                rb = (lo + 2 * r) * F
                return tuple(
                    jnp.maximum(
                        m[v],
                        jnp.maximum(bufx[pl.ds(rb + v * 16, 16)],
                                    bufx[pl.ds(rb + F + v * 16, 16)]),
                    )
                    for v in range(NV)
                )

            m = lax.fori_loop(0, nh, max_body, minf8)
            m = tuple(
                jnp.maximum(m[v], bufx[pl.ds(rbt + v * 16, 16)])
                for v in range(NV)
            )

            def exp_body(r, s):
                rb = (lo + 2 * r) * F
                out = []
                for v in range(NV):
                    e0 = jnp.exp(bufx[pl.ds(rb + v * 16, 16)] - m[v])
                    e1 = jnp.exp(bufx[pl.ds(rb + F + v * 16, 16)] - m[v])
                    bufx[pl.ds(rb + v * 16, 16)] = e0
                    bufx[pl.ds(rb + F + v * 16, 16)] = e1
                    out.append(s[v] + e0 + e1)
                return tuple(out)

            ssum = lax.fori_loop(0, nh, exp_body, zero8)

            @pl.when(odd)
            def _():
                for v in range(NV):
                    bufx[pl.ds(rbt + v * 16, 16)] = jnp.exp(
                        bufx[pl.ds(rbt + v * 16, 16)] - m[v]
                    )

            ssum = tuple(
                ssum[v]
                + jnp.where(odd, bufx[pl.ds(rbt + v * 16, 16)], 0.0)
                for v in range(NV)
            )
            rcp = tuple(1.0 / ssum[v] for v in range(NV))

            def scale_body(r, carry):
                rb = (lo + 2 * r) * F
                for v in range(NV):
                    bufx[pl.ds(rb + v * 16, 16)] = (
                        bufx[pl.ds(rb + v * 16, 16)] * rcp[v]
                    )
                    bufx[pl.ds(rb + F + v * 16, 16)] = (
                        bufx[pl.ds(rb + F + v * 16, 16)] * rcp[v]
                    )
                return carry

            lax.fori_loop(0, nh, scale_body, 0)

            @pl.when(odd)
            def _():
                for v in range(NV):
                    bufx[pl.ds(rbt + v * 16, 16)] = (
                        bufx[pl.ds(rbt + v * 16, 16)] * rcp[v]
                    )

        def seg_stream(bufx, st, en):
            n = en - st
            nch = (n + CH - 1) // CH

            def chunk_a(i, m):
                cb = st + i * CH
                csz = jnp.minimum(CH, en - cb)
                _copy_rows(data_hbm, bufx, cb, 0, csz, 9, dsem)

                def mb(r, mm):
                    rb = r * F
                    return tuple(
                        jnp.maximum(mm[v], bufx[pl.ds(rb + v * 16, 16)])
                        for v in range(NV)
                    )

                return lax.fori_loop(0, csz, mb, m)

            m = lax.fori_loop(0, nch, chunk_a, minf8)

            def chunk_b(i, s):
                cb = st + i * CH
                csz = jnp.minimum(CH, en - cb)
                _copy_rows(data_hbm, bufx, cb, 0, csz, 9, dsem)

                def eb(r, ss):
                    rb = r * F
                    out = []
                    for v in range(NV):
                        e = jnp.exp(bufx[pl.ds(rb + v * 16, 16)] - m[v])
                        bufx[pl.ds(rb + v * 16, 16)] = e
                        out.append(ss[v] + e)
                    return tuple(out)

                s2 = lax.fori_loop(0, csz, eb, s)
                _copy_rows(bufx, out_hbm, 0, cb, csz, 9, dsem)
                return s2

            ssum = lax.fori_loop(0, nch, chunk_b, zero8)
            rcp = tuple(1.0 / ssum[v] for v in range(NV))

            def chunk_c(i, carry):
                cb = st + i * CH
                csz = jnp.minimum(CH, en - cb)
                _copy_rows(out_hbm, bufx, cb, 0, csz, 9, dsem)

                def sb(r, cc):
                    rb = r * F
                    for v in range(NV):
                        bufx[pl.ds(rb + v * 16, 16)] = (
                            bufx[pl.ds(rb + v * 16, 16)] * rcp[v]
                        )
                    return cc

                lax.fori_loop(0, csz, sb, 0)
                _copy_rows(bufx, out_hbm, 0, cb, csz, 9, dsem)
                return carry

            lax.fori_loop(0, nch, chunk_c, 0)

        ngroups = (ns + G - 1) // G

        def g_info(g):
            kidx = g * G
            kend = jnp.minimum(kidx + G, ns)
            return kidx, kend, bd_read(kidx), bd_read(kend)

        def compute_fast(g, bufx, row0):
            kidx, kend, _, _ = g_info(g)

            @pl.loop(0, G)
            def _(j):
                @pl.when(kidx + j < kend)
                def _():
                    st = bd_read(kidx + j)
                    en = bd_read(kidx + j + 1)
                    seg_compute(bufx, st - row0, en - st)

        def fallback(g, bufx):
            kidx, kend, _, _ = g_info(g)

            @pl.loop(0, G)
            def _(j):
                @pl.when(kidx + j < kend)
                def _():
                    st = bd_read(kidx + j)
                    en = bd_read(kidx + j + 1)
                    seg_stream(bufx, st, en)

        npairs = (ngroups + 1) // 2

        @pl.loop(0, npairs)
        def _(p):
            g0 = p * 2
            g1 = g0 + 1
            _, _, r00, r01 = g_info(g0)
            ng0 = r01 - r00
            in1 = g1 < ngroups
            _, _, r10, r11 = g_info(g1)
            ng1 = r11 - r10
            ld1 = in1 & (ng1 <= C)

            @pl.when(ng0 <= C)
            def _():
                _issue_rows(data_hbm, bufa, r00, 0, ng0, 9, lsem0)

            @pl.when(ld1)
            def _():
                _issue_rows(data_hbm, bufb, r10, 0, ng1, 9, lsem1)

            @pl.when(ng0 <= C)
            def _():
                _wait_rows(data_hbm, bufa, r00, 0, ng0, 9, lsem0)
                compute_fast(g0, bufa, r00)
                _issue_rows(bufa, out_hbm, 0, r00, ng0, 9, ssem)

            @pl.when(ng0 > C)
            def _():
                fallback(g0, bufa)

            @pl.when(ld1)
            def _():
                _wait_rows(data_hbm, bufb, r10, 0, ng1, 9, lsem1)
                compute_fast(g1, bufb, r10)

            @pl.when(ng0 <= C)
            def _():
                _wait_rows(bufa, out_hbm, 0, r00, ng0, 9, ssem)

            @pl.when(ld1)
            def _():
                _copy_rows(bufb, out_hbm, 0, r10, ng1, 9, dsem)

            @pl.when(in1 & (ng1 > C))
            def _():
                fallback(g1, bufb)

    return body(data_flat, ids_pad)


@jax.jit
def kernel(data, segment_ids):
    ids = segment_ids.astype(jnp.int32)
    head = jnp.full((8,), -1, jnp.int32)
    tail = jnp.full((IDS_PAD - 8 - N,), BIG, jnp.int32)
    ids_pad = jnp.concatenate([head, ids, tail])
    out_flat = _sc_softmax(data.reshape(-1), ids_pad)
    return out_flat.reshape(N, F)
